# baseline (speedup 1.0000x reference)
import functools
import math

import jax
import jax.numpy as jnp
from jax import lax
from jax.experimental import pallas as pl
from jax.experimental.pallas import tpu as pltpu
from jax.experimental.pallas import tpu_sc as plsc

_D = 512
_NC = 2
_NS = 16
_NW = _NC * _NS
_CHUNK = 120
_LANES = 16


def _sc_normalize(descriptors, a, b):
    n, d = descriptors.shape
    oct_total = n // 8
    oct_per = oct_total // _NW
    oct_rem = oct_total % _NW
    n_chunks = (oct_per * 8) // _CHUNK
    assert (oct_per * 8) % _CHUNK == 0 and n_chunks % 2 == 0
    mesh = plsc.VectorSubcoreMesh(core_axis_name="c", subcore_axis_name="s")

    @functools.partial(
        pl.kernel,
        out_type=jax.ShapeDtypeStruct((n, d), jnp.float32),
        mesh=mesh,
        scratch_types=[
            pltpu.VMEM((_CHUNK, _D), jnp.float32),
            pltpu.VMEM((_CHUNK, _D), jnp.float32),
            pltpu.VMEM((_D,), jnp.float32),
            pltpu.VMEM((_D,), jnp.float32),
            pltpu.SemaphoreType.DMA,
            pltpu.SemaphoreType.DMA,
            pltpu.SemaphoreType.DMA,
            pltpu.SemaphoreType.DMA,
        ],
    )
    def k(x_hbm, a_hbm, b_hbm, o_hbm, buf0, buf1, a_v, b_v, si0, si1, so0, so1):
        wid = lax.axis_index("s") * _NC + lax.axis_index("c")
        base = (wid * oct_per + jnp.minimum(wid, oct_rem)) * 8

        pltpu.sync_copy(a_hbm, a_v)
        pltpu.sync_copy(b_hbm, b_v)

        def row0(g, rows=_CHUNK):
            return pl.multiple_of(base + g * rows, 8)

        def start_in(g, buf, sem):
            pltpu.async_copy(x_hbm.at[pl.ds(row0(g), _CHUNK)], buf, sem)

        def start_out(g, buf, sem):
            pltpu.async_copy(buf, o_hbm.at[pl.ds(row0(g), _CHUNK)], sem)

        def wait_in(buf, sem):
            pltpu.make_async_copy(x_hbm.at[pl.ds(0, _CHUNK)], buf, sem).wait()

        def wait_out(buf, sem):
            pltpu.make_async_copy(buf, o_hbm.at[pl.ds(0, _CHUNK)], sem).wait()

        def compute(buf, rows):
            unroll = 8
            for j in range(_D // _LANES):
                sl = pl.ds(j * _LANES, _LANES)
                aj = a_v[sl]
                bj = b_v[sl]

                def rbody(r, _, sl=sl, aj=aj, bj=bj):
                    for u in range(unroll):
                        buf[r * unroll + u, sl] = buf[r * unroll + u, sl] * aj + bj
                    return 0

                lax.fori_loop(0, rows // unroll, rbody, 0)

        start_in(0, buf0, si0)
        start_in(1, buf1, si1)

        def phase(g, buf, s_in, s_out):
            wait_in(buf, s_in)
            compute(buf, _CHUNK)
            start_out(g, buf, s_out)

            @pl.when(g + 2 < n_chunks)
            def _():
                wait_out(buf, s_out)
                start_in(g + 2, buf, s_in)

        def pair_body(p, _):
            phase(2 * p, buf0, si0, so0)
            phase(2 * p + 1, buf1, si1, so1)
            return 0

        lax.fori_loop(0, n_chunks // 2, pair_body, 0)
        wait_out(buf0, so0)
        wait_out(buf1, so1)

        @pl.when(wid < oct_rem)
        def _():
            tail = pl.multiple_of(base + n_chunks * _CHUNK, 8)
            pltpu.async_copy(x_hbm.at[pl.ds(tail, 8)], buf0.at[pl.ds(0, 8)], si0)
            pltpu.make_async_copy(
                x_hbm.at[pl.ds(0, 8)], buf0.at[pl.ds(0, 8)], si0).wait()
            compute(buf0, 8)
            pltpu.async_copy(buf0.at[pl.ds(0, 8)], o_hbm.at[pl.ds(tail, 8)], so0)
            pltpu.make_async_copy(
                buf0.at[pl.ds(0, 8)], o_hbm.at[pl.ds(0, 8)], so0).wait()

    return k(descriptors, a, b)


def kernel(descriptors, mean, std):
    n, d = descriptors.shape
    a = (1.0 / (std * math.sqrt(d))).reshape(d)
    b = (-mean).reshape(d) * a
    return _sc_normalize(descriptors, a, b)
